# baseline (speedup 1.0000x reference)
import functools

import jax
import jax.numpy as jnp
import numpy as np
from jax import lax
from jax.experimental import pallas as pl
from jax.experimental.pallas import tpu as pltpu
from jax.experimental.pallas import tpu_sc as plsc

_HID = 8
_STEPS = 3
_NG = 64
_PK = 8

_R8 = np.kron(np.eye(_HID), np.ones((1, _HID))).astype(np.float32)
_F8 = np.kron(np.ones((_HID, 1)), np.eye(_HID)).astype(np.float32)
_RBD = np.kron(np.eye(_PK), _R8).astype(np.float32)
_FBD = np.kron(np.eye(_PK), _F8).astype(np.float32)


def _pre_body(x_ref, wp_ref, bp_ref, o_ref):
    o_ref[...] = jnp.maximum(
        jnp.dot(x_ref[...], wp_ref[...],
                preferred_element_type=jnp.float32) + bp_ref[...], 0.0)


def _msg_body(ea_ref, hs_ref, w1_ref, b1_ref, w2_ref, b2_ref, r_ref, f_ref,
              o_ref):
    eh = jnp.maximum(
        jnp.dot(ea_ref[...], w1_ref[...],
                preferred_element_type=jnp.float32) + b1_ref[...], 0.0)
    ew = jnp.dot(eh, w2_ref[...],
                 preferred_element_type=jnp.float32) + b2_ref[...]
    hrep = jnp.dot(hs_ref[...], r_ref[...],
                   preferred_element_type=jnp.float32)
    o_ref[...] = jnp.dot(hrep * ew, f_ref[...],
                         preferred_element_type=jnp.float32)


def _node_body(aggp_ref, h_ref, wr_ref, bc_ref, wih_ref, bih_ref,
               whh_ref, bhh_ref, o_ref):
    agg = aggp_ref[0] + aggp_ref[1]
    h = h_ref[...]
    m = jnp.maximum(
        agg + jnp.dot(h, wr_ref[...], preferred_element_type=jnp.float32)
        + bc_ref[...], 0.0)
    gi = jnp.dot(m, wih_ref[...], preferred_element_type=jnp.float32) \
        + bih_ref[...]
    gh = jnp.dot(h, whh_ref[...], preferred_element_type=jnp.float32) \
        + bhh_ref[...]
    r = jax.nn.sigmoid(gi[:, 0:_HID] + gh[:, 0:_HID])
    z = jax.nn.sigmoid(gi[:, _HID:2 * _HID] + gh[:, _HID:2 * _HID])
    n = jnp.tanh(gi[:, 2 * _HID:] + r * gh[:, 2 * _HID:])
    o_ref[...] = (1.0 - z) * n + z * h


def _read_body(h_ref, b_ref, w1_ref, b1_ref, w2_ref, b2_ref, wp_ref, bp_ref,
               o_ref):
    h = h_ref[...]
    nf = jnp.maximum(
        jnp.dot(h, w1_ref[...], preferred_element_type=jnp.float32)
        + b1_ref[...], 0.0)
    nf = jnp.dot(nf, w2_ref[...], preferred_element_type=jnp.float32) \
        + b2_ref[...]
    n = h.shape[0]
    oh = (b_ref[...] == lax.broadcasted_iota(jnp.int32, (n, _NG), 1)
          ).astype(jnp.float32)
    dn = (((0,), (0,)), ((), ()))
    sums = lax.dot_general(oh, nf, dn, preferred_element_type=jnp.float32)
    counts = lax.dot_general(oh, jnp.ones((n, 1), jnp.float32), dn,
                             preferred_element_type=jnp.float32)
    g = sums / jnp.maximum(counts, 1.0)
    o_ref[...] = jnp.dot(g, wp_ref[...],
                         preferred_element_type=jnp.float32) + bp_ref[...]


def _make_sc_gather(n, e, nc, ns):
    nw = nc * ns
    bpw = e // nw
    mesh = plsc.VectorSubcoreMesh(core_axis_name="c", subcore_axis_name="s")

    @functools.partial(
        pl.kernel, mesh=mesh,
        out_type=jax.ShapeDtypeStruct((e, _HID), jnp.float32),
        scratch_types=[
            pltpu.VMEM((bpw,), jnp.int32),
            pltpu.VMEM((bpw, _HID), jnp.float32),
            pltpu.SemaphoreType.DMA,
        ],
        compiler_params=pltpu.CompilerParams(use_tc_tiling_on_sc=False),
    )
    def sc_gather(h_hbm, src_hbm, out_hbm, idx_v, rows_v, sem):
        wid = lax.axis_index("s") * nc + lax.axis_index("c")
        base = wid * bpw
        pltpu.sync_copy(src_hbm.at[pl.ds(base, bpw)], idx_v)
        pltpu.async_copy(h_hbm.at[idx_v], rows_v, sem).wait()
        pltpu.sync_copy(rows_v, out_hbm.at[pl.ds(base, bpw)])

    return sc_gather


def _make_sc_scatter(n, e, nc, ns):
    nw = nc * ns
    bpw = e // nw
    mesh = plsc.VectorSubcoreMesh(core_axis_name="c", subcore_axis_name="s")

    @functools.partial(
        pl.kernel, mesh=mesh,
        out_type=jax.ShapeDtypeStruct((nc, n, _HID), jnp.float32),
        scratch_types=[
            pltpu.VMEM((bpw,), jnp.int32),
            pltpu.VMEM((bpw, _HID), jnp.float32),
            pltpu.VMEM_SHARED((n, _HID), jnp.float32),
        ],
        compiler_params=pltpu.CompilerParams(use_tc_tiling_on_sc=False),
    )
    def sc_scatter(msg_hbm, dst_hbm, zeros_hbm, out_hbm, idx_v, msg_v,
                   agg_sh):
        cid = lax.axis_index("c")
        sid = lax.axis_index("s")
        wid = sid * nc + cid
        base = wid * bpw

        @pl.when(sid == 0)
        def _():
            pltpu.sync_copy(zeros_hbm, agg_sh)

        plsc.subcore_barrier()
        pltpu.sync_copy(dst_hbm.at[pl.ds(base, bpw)], idx_v)
        pltpu.sync_copy(msg_hbm.at[pl.ds(base, bpw)], msg_v)
        pltpu.sync_copy(msg_v, agg_sh.at[idx_v], add=True)
        plsc.subcore_barrier()

        @pl.when(sid == 0)
        def _():
            pltpu.sync_copy(agg_sh, out_hbm.at[cid])

    return sc_scatter


def kernel(x, edge_index, edge_attr, batch, W_proj, b_proj, W_e1, b_e1,
           W_e2, b_e2, W_root, b_conv, W_gru_ih, b_gru_ih, W_gru_hh,
           b_gru_hh, W_r1, b_r1, W_r2, b_r2, W_p, b_p):
    n, df = x.shape
    e = edge_attr.shape[0]
    de = edge_attr.shape[1]
    src = edge_index[0].astype(jnp.int32)
    dst = edge_index[1].astype(jnp.int32)

    info = plsc.get_sparse_core_info()
    nc, ns = info.num_cores, info.num_subcores
    sc_gather = _make_sc_gather(n, e, nc, ns)
    sc_scatter = _make_sc_scatter(n, e, nc, ns)

    h = pl.pallas_call(
        _pre_body,
        out_shape=jax.ShapeDtypeStruct((n, _HID), jnp.float32),
    )(x, W_proj, b_proj.reshape(1, _HID))

    rows = e // _PK
    rb = 4000
    grid = rows // rb
    msg_call = pl.pallas_call(
        _msg_body,
        grid=(grid,),
        in_specs=[
            pl.BlockSpec((rb, _PK * de), lambda i: (i, 0)),
            pl.BlockSpec((rb, _PK * _HID), lambda i: (i, 0)),
            pl.BlockSpec((_PK * de, _PK * 16), lambda i: (0, 0)),
            pl.BlockSpec((1, _PK * 16), lambda i: (0, 0)),
            pl.BlockSpec((_PK * 16, _PK * 64), lambda i: (0, 0)),
            pl.BlockSpec((1, _PK * 64), lambda i: (0, 0)),
            pl.BlockSpec((_PK * _HID, _PK * 64), lambda i: (0, 0)),
            pl.BlockSpec((_PK * 64, _PK * _HID), lambda i: (0, 0)),
        ],
        out_specs=pl.BlockSpec((rb, _PK * _HID), lambda i: (i, 0)),
        out_shape=jax.ShapeDtypeStruct((rows, _PK * _HID), jnp.float32),
    )
    w1bd = jnp.kron(jnp.eye(_PK, dtype=jnp.float32), W_e1)
    b1t = jnp.tile(b_e1, _PK).reshape(1, _PK * 16)
    w2bd = jnp.kron(jnp.eye(_PK, dtype=jnp.float32), W_e2)
    b2t = jnp.tile(b_e2, _PK).reshape(1, _PK * 64)
    rbd = jnp.asarray(_RBD)
    fbd = jnp.asarray(_FBD)
    ea_p = edge_attr.reshape(rows, _PK * de)
    node_call = pl.pallas_call(
        _node_body,
        out_shape=jax.ShapeDtypeStruct((n, _HID), jnp.float32),
    )

    zeros = jnp.zeros((n, _HID), jnp.float32)
    for _ in range(_STEPS):
        h_src = sc_gather(h, src)
        msg_p = msg_call(ea_p, h_src.reshape(rows, _PK * _HID), w1bd, b1t,
                         w2bd, b2t, rbd, fbd)
        aggp = sc_scatter(msg_p.reshape(e, _HID), dst, zeros)
        h = node_call(aggp, h, W_root, b_conv.reshape(1, _HID),
                      W_gru_ih, b_gru_ih.reshape(1, 3 * _HID),
                      W_gru_hh, b_gru_hh.reshape(1, 3 * _HID))

    out = pl.pallas_call(
        _read_body,
        out_shape=jax.ShapeDtypeStruct((_NG, 1), jnp.float32),
    )(h, batch.astype(jnp.int32).reshape(n, 1), W_r1, b_r1.reshape(1, _HID),
      W_r2, b_r2.reshape(1, _HID), W_p, b_p.reshape(1, 1))
    return out


# speedup vs baseline: 1.0577x; 1.0577x over previous
import functools

import jax
import jax.numpy as jnp
import numpy as np
from jax import lax
from jax.experimental import pallas as pl
from jax.experimental.pallas import tpu as pltpu
from jax.experimental.pallas import tpu_sc as plsc

_HID = 8
_STEPS = 3
_NG = 64
_PK = 8

_R8 = np.kron(np.eye(_HID), np.ones((1, _HID))).astype(np.float32)
_F8 = np.kron(np.ones((_HID, 1)), np.eye(_HID)).astype(np.float32)
_RBD = np.kron(np.eye(_PK), _R8).astype(np.float32)
_FBD = np.kron(np.eye(_PK), _F8).astype(np.float32)


def _pre_body(x_ref, wp_ref, bp_ref, o_ref):
    o_ref[...] = jnp.maximum(
        jnp.dot(x_ref[...], wp_ref[...],
                preferred_element_type=jnp.float32) + bp_ref[...], 0.0)


def _msg_body(ea_ref, hs_ref, w1_ref, b1_ref, w2_ref, b2_ref, r_ref, f_ref,
              o_ref):
    eh = jnp.maximum(
        jnp.dot(ea_ref[...], w1_ref[...],
                preferred_element_type=jnp.float32) + b1_ref[...], 0.0)
    ew = jnp.dot(eh, w2_ref[...],
                 preferred_element_type=jnp.float32) + b2_ref[...]
    hrep = jnp.dot(hs_ref[...], r_ref[...],
                   preferred_element_type=jnp.float32)
    o_ref[...] = jnp.dot(hrep * ew, f_ref[...],
                         preferred_element_type=jnp.float32)


def _node_body(aggp_ref, h_ref, wr_ref, bc_ref, wih_ref, bih_ref,
               whh_ref, bhh_ref, o_ref):
    agg = aggp_ref[0] + aggp_ref[1]
    h = h_ref[...]
    m = jnp.maximum(
        agg + jnp.dot(h, wr_ref[...], preferred_element_type=jnp.float32)
        + bc_ref[...], 0.0)
    gi = jnp.dot(m, wih_ref[...], preferred_element_type=jnp.float32) \
        + bih_ref[...]
    gh = jnp.dot(h, whh_ref[...], preferred_element_type=jnp.float32) \
        + bhh_ref[...]
    r = jax.nn.sigmoid(gi[:, 0:_HID] + gh[:, 0:_HID])
    z = jax.nn.sigmoid(gi[:, _HID:2 * _HID] + gh[:, _HID:2 * _HID])
    n = jnp.tanh(gi[:, 2 * _HID:] + r * gh[:, 2 * _HID:])
    o_ref[...] = (1.0 - z) * n + z * h


def _read_body(h_ref, b_ref, w1_ref, b1_ref, w2_ref, b2_ref, wp_ref, bp_ref,
               o_ref):
    h = h_ref[...]
    nf = jnp.maximum(
        jnp.dot(h, w1_ref[...], preferred_element_type=jnp.float32)
        + b1_ref[...], 0.0)
    nf = jnp.dot(nf, w2_ref[...], preferred_element_type=jnp.float32) \
        + b2_ref[...]
    n = h.shape[0]
    oh = (b_ref[...] == lax.broadcasted_iota(jnp.int32, (n, _NG), 1)
          ).astype(jnp.float32)
    dn = (((0,), (0,)), ((), ()))
    sums = lax.dot_general(oh, nf, dn, preferred_element_type=jnp.float32)
    counts = lax.dot_general(oh, jnp.ones((n, 1), jnp.float32), dn,
                             preferred_element_type=jnp.float32)
    g = sums / jnp.maximum(counts, 1.0)
    o_ref[...] = jnp.dot(g, wp_ref[...],
                         preferred_element_type=jnp.float32) + bp_ref[...]


def _make_sc_gather(n, e, nc, ns):
    nw = nc * ns
    bpw = e // nw
    mesh = plsc.VectorSubcoreMesh(core_axis_name="c", subcore_axis_name="s")

    @functools.partial(
        pl.kernel, mesh=mesh,
        out_type=jax.ShapeDtypeStruct((e, _HID), jnp.float32),
        scratch_types=[
            pltpu.VMEM((bpw,), jnp.int32),
            pltpu.VMEM((bpw, _HID), jnp.float32),
            pltpu.VMEM_SHARED((n, _HID), jnp.float32),
            pltpu.SemaphoreType.DMA,
        ],
        compiler_params=pltpu.CompilerParams(use_tc_tiling_on_sc=False),
    )
    def sc_gather(h_hbm, src_hbm, out_hbm, idx_v, rows_v, h_sh, sem):
        cid = lax.axis_index("c")
        sid = lax.axis_index("s")
        wid = sid * nc + cid
        base = wid * bpw

        @pl.when(sid == 0)
        def _():
            pltpu.sync_copy(h_hbm, h_sh)

        pltpu.sync_copy(src_hbm.at[pl.ds(base, bpw)], idx_v)
        plsc.subcore_barrier()
        pltpu.async_copy(h_sh.at[idx_v], rows_v, sem).wait()
        pltpu.sync_copy(rows_v, out_hbm.at[pl.ds(base, bpw)])

    return sc_gather


def _make_sc_scatter(n, e, nc, ns):
    nw = nc * ns
    bpw = e // nw
    mesh = plsc.VectorSubcoreMesh(core_axis_name="c", subcore_axis_name="s")

    @functools.partial(
        pl.kernel, mesh=mesh,
        out_type=jax.ShapeDtypeStruct((nc, n, _HID), jnp.float32),
        scratch_types=[
            pltpu.VMEM((bpw,), jnp.int32),
            pltpu.VMEM((bpw, _HID), jnp.float32),
            pltpu.VMEM_SHARED((n, _HID), jnp.float32),
        ],
        compiler_params=pltpu.CompilerParams(use_tc_tiling_on_sc=False),
    )
    def sc_scatter(msg_hbm, dst_hbm, zeros_hbm, out_hbm, idx_v, msg_v,
                   agg_sh):
        cid = lax.axis_index("c")
        sid = lax.axis_index("s")
        wid = sid * nc + cid
        base = wid * bpw

        @pl.when(sid == 0)
        def _():
            pltpu.sync_copy(zeros_hbm, agg_sh)

        plsc.subcore_barrier()
        pltpu.sync_copy(dst_hbm.at[pl.ds(base, bpw)], idx_v)
        pltpu.sync_copy(msg_hbm.at[pl.ds(base, bpw)], msg_v)
        pltpu.sync_copy(msg_v, agg_sh.at[idx_v], add=True)
        plsc.subcore_barrier()

        @pl.when(sid == 0)
        def _():
            pltpu.sync_copy(agg_sh, out_hbm.at[cid])

    return sc_scatter


def kernel(x, edge_index, edge_attr, batch, W_proj, b_proj, W_e1, b_e1,
           W_e2, b_e2, W_root, b_conv, W_gru_ih, b_gru_ih, W_gru_hh,
           b_gru_hh, W_r1, b_r1, W_r2, b_r2, W_p, b_p):
    n, df = x.shape
    e = edge_attr.shape[0]
    de = edge_attr.shape[1]
    src = edge_index[0].astype(jnp.int32)
    dst = edge_index[1].astype(jnp.int32)

    info = plsc.get_sparse_core_info()
    nc, ns = info.num_cores, info.num_subcores
    sc_gather = _make_sc_gather(n, e, nc, ns)
    sc_scatter = _make_sc_scatter(n, e, nc, ns)

    h = pl.pallas_call(
        _pre_body,
        out_shape=jax.ShapeDtypeStruct((n, _HID), jnp.float32),
    )(x, W_proj, b_proj.reshape(1, _HID))

    rows = e // _PK
    rb = 2000
    grid = rows // rb
    msg_call = pl.pallas_call(
        _msg_body,
        grid=(grid,),
        in_specs=[
            pl.BlockSpec((rb, _PK * de), lambda i: (i, 0)),
            pl.BlockSpec((rb, _PK * _HID), lambda i: (i, 0)),
            pl.BlockSpec((_PK * de, _PK * 16), lambda i: (0, 0)),
            pl.BlockSpec((1, _PK * 16), lambda i: (0, 0)),
            pl.BlockSpec((_PK * 16, _PK * 64), lambda i: (0, 0)),
            pl.BlockSpec((1, _PK * 64), lambda i: (0, 0)),
            pl.BlockSpec((_PK * _HID, _PK * 64), lambda i: (0, 0)),
            pl.BlockSpec((_PK * 64, _PK * _HID), lambda i: (0, 0)),
        ],
        out_specs=pl.BlockSpec((rb, _PK * _HID), lambda i: (i, 0)),
        out_shape=jax.ShapeDtypeStruct((rows, _PK * _HID), jnp.float32),
    )
    w1bd = jnp.kron(jnp.eye(_PK, dtype=jnp.float32), W_e1)
    b1t = jnp.tile(b_e1, _PK).reshape(1, _PK * 16)
    w2bd = jnp.kron(jnp.eye(_PK, dtype=jnp.float32), W_e2)
    b2t = jnp.tile(b_e2, _PK).reshape(1, _PK * 64)
    rbd = jnp.asarray(_RBD)
    fbd = jnp.asarray(_FBD)
    ea_p = edge_attr.reshape(rows, _PK * de)
    node_call = pl.pallas_call(
        _node_body,
        out_shape=jax.ShapeDtypeStruct((n, _HID), jnp.float32),
    )

    zeros = jnp.zeros((n, _HID), jnp.float32)
    for _ in range(_STEPS):
        h_src = sc_gather(h, src)
        msg_p = msg_call(ea_p, h_src.reshape(rows, _PK * _HID), w1bd, b1t,
                         w2bd, b2t, rbd, fbd)
        aggp = sc_scatter(msg_p.reshape(e, _HID), dst, zeros)
        h = node_call(aggp, h, W_root, b_conv.reshape(1, _HID),
                      W_gru_ih, b_gru_ih.reshape(1, 3 * _HID),
                      W_gru_hh, b_gru_hh.reshape(1, 3 * _HID))

    out = pl.pallas_call(
        _read_body,
        out_shape=jax.ShapeDtypeStruct((_NG, 1), jnp.float32),
    )(h, batch.astype(jnp.int32).reshape(n, 1), W_r1, b_r1.reshape(1, _HID),
      W_r2, b_r2.reshape(1, _HID), W_p, b_p.reshape(1, 1))
    return out
